# baseline (speedup 1.0000x reference)
import functools

import jax
import jax.numpy as jnp
from jax import lax
from jax.experimental import pallas as pl
from jax.experimental.pallas import tpu as pltpu
from jax.experimental.pallas import tpu_sc as plsc

_ROWS = 16384
_COLS = 256
_BLOCK_ROWS = 7680

_NC, _NS = 2, 16
_NW = _NC * _NS
_W_ROWS = _ROWS // _NW
_CH_ROWS = 128
_N_CH = _W_ROWS // _CH_ROWS


def _tc_copy_body(t_ref, t_out):
    t_out[...] = t_ref[...]


def _sc_copy_body(q_hbm, q_out, buf_a, buf_b, sems):
    wid = lax.axis_index("s") * _NC + lax.axis_index("c")
    base = wid * _W_ROWS

    jobs = []
    for c in range(_N_CH):
        sl = pl.ds(base + c * _CH_ROWS, _CH_ROWS)
        jobs.append((q_hbm.at[sl], q_out.at[sl]))

    bufs = (buf_a, buf_b)
    n = len(jobs)
    loads = [None] * n
    stores = [None] * n
    loads[0] = pltpu.async_copy(jobs[0][0], bufs[0], sems.at[0])
    for j in range(n):
        b = j % 2
        loads[j].wait()
        stores[j] = pltpu.async_copy(bufs[b], jobs[j][1], sems.at[2 + b])
        if j + 1 < n:
            if j - 1 >= 0:
                stores[j - 1].wait()
            loads[j + 1] = pltpu.async_copy(
                jobs[j + 1][0], bufs[(j + 1) % 2], sems.at[(j + 1) % 2]
            )
    stores[n - 2].wait()
    stores[n - 1].wait()


def kernel(emb_targets, emb_queries):
    grid = (-(-_ROWS // _BLOCK_ROWS),)
    spec = pl.BlockSpec((_BLOCK_ROWS, _COLS), lambda i: (i, 0))
    out_t = pl.pallas_call(
        _tc_copy_body,
        grid=grid,
        in_specs=[spec],
        out_specs=spec,
        out_shape=jax.ShapeDtypeStruct((_ROWS, _COLS), jnp.float32),
        compiler_params=pltpu.CompilerParams(
            vmem_limit_bytes=100 * 1024 * 1024, dimension_semantics=("parallel",)
        ),
    )(emb_targets)

    mesh = plsc.VectorSubcoreMesh(
        core_axis_name="c", subcore_axis_name="s", num_cores=_NC, num_subcores=_NS
    )
    sc_copy = functools.partial(
        pl.kernel,
        mesh=mesh,
        out_type=jax.ShapeDtypeStruct((_ROWS, _COLS), jnp.float32),
        scratch_types=[
            pltpu.VMEM((_CH_ROWS, _COLS), jnp.float32),
            pltpu.VMEM((_CH_ROWS, _COLS), jnp.float32),
            pltpu.SemaphoreType.DMA((4,)),
        ],
    )(_sc_copy_body)
    out_q = sc_copy(emb_queries)
    return (out_t, out_q)


# speedup vs baseline: 1.9129x; 1.9129x over previous
import jax
import jax.numpy as jnp
from jax.experimental import pallas as pl
from jax.experimental.pallas import tpu as pltpu

_ROWS = 16384
_COLS = 256
_BLOCK_ROWS = 7680


def _copy_body(t_ref, q_ref, t_out, q_out):
    t_out[...] = t_ref[...]
    q_out[...] = q_ref[...]


def kernel(emb_targets, emb_queries):
    grid = (-(-_ROWS // _BLOCK_ROWS),)
    spec = pl.BlockSpec((_BLOCK_ROWS, _COLS), lambda i: (i, 0))
    out_t, out_q = pl.pallas_call(
        _copy_body,
        grid=grid,
        in_specs=[spec, spec],
        out_specs=[spec, spec],
        out_shape=[
            jax.ShapeDtypeStruct((_ROWS, _COLS), jnp.float32),
            jax.ShapeDtypeStruct((_ROWS, _COLS), jnp.float32),
        ],
        compiler_params=pltpu.CompilerParams(
            vmem_limit_bytes=100 * 1024 * 1024, dimension_semantics=("parallel",)
        ),
    )(emb_targets, emb_queries)
    return (out_t, out_q)
